# baseline (speedup 1.0000x reference)
import functools

import jax
import jax.numpy as jnp
from jax import lax
from jax.experimental import pallas as pl
from jax.experimental.pallas import tpu as pltpu
from jax.experimental.pallas import tpu_sc as plsc

N = 10000
NP = 10240
F = 256
HF = 128
FC1 = 128
E = 160000
EP = 163840
G = 64
DUMMY = 10200
RB = 1024
GRID = NP // RB
CH = 128
CHA = 64
NCH_AGG = (EP // 16) // CHA
QTR_A = NCH_AGG // 4
NCH_DEG = (EP // 32) // CH
ROWS_PT = NP // 16
NZ = 10112
ZROWS_PT = NZ // 16


def _sc_degree(dst32, e0rows, zeros_d):
    mesh = plsc.VectorSubcoreMesh(core_axis_name="c", subcore_axis_name="s")

    @functools.partial(
        pl.kernel,
        out_type=(jax.ShapeDtypeStruct((NP, 16), jnp.float32),
                  jax.ShapeDtypeStruct((NP, 16), jnp.float32)),
        mesh=mesh,
        scratch_types=[
            pltpu.VMEM((NCH_DEG, CH), jnp.int32),
            pltpu.VMEM((CH, 16), jnp.float32),
            pltpu.VMEM_SHARED((NP, 16), jnp.float32),
        ],
    )
    def deg_kernel(dst_hbm, e0_hbm, zd_hbm, d0_hbm, d1_hbm, dstv, e0v, deg_sp):
        cid = lax.axis_index("c")
        sid = lax.axis_index("s")
        w = cid * 16 + sid
        sl = pl.ds(sid * ROWS_PT, ROWS_PT)
        pltpu.sync_copy(zd_hbm, deg_sp.at[sl])
        pltpu.sync_copy(e0_hbm, e0v)
        pltpu.sync_copy(dst_hbm.at[w], dstv)
        plsc.subcore_barrier()

        def body(i, carry):
            pltpu.sync_copy(e0v, deg_sp.at[dstv.at[i]], add=True)
            return carry

        lax.fori_loop(0, NCH_DEG, body, 0)
        plsc.subcore_barrier()

        @pl.when(cid == 0)
        def _():
            pltpu.sync_copy(deg_sp.at[sl], d0_hbm.at[sl])

        @pl.when(cid == 1)
        def _():
            pltpu.sync_copy(deg_sp.at[sl], d1_hbm.at[sl])

    return deg_kernel(dst32, e0rows, zeros_d)


def _sc_aggregate(yl, yr, src16, dst16, zeros_z):
    mesh = plsc.VectorSubcoreMesh(core_axis_name="c", subcore_axis_name="s")

    @functools.partial(
        pl.kernel,
        out_type=(jax.ShapeDtypeStruct((NP, HF), jnp.float32),
                  jax.ShapeDtypeStruct((NP, HF), jnp.float32)),
        mesh=mesh,
        scratch_types=[
            pltpu.VMEM((QTR_A, CHA), jnp.int32),
            pltpu.VMEM((QTR_A, CHA), jnp.int32),
            pltpu.VMEM((CHA, HF), jnp.float32),
            pltpu.VMEM((CHA, HF), jnp.float32),
            pltpu.VMEM((CHA, HF), jnp.float32),
            pltpu.VMEM((CHA, HF), jnp.float32),
            pltpu.VMEM_SHARED((NZ, HF), jnp.float32),
            pltpu.SemaphoreType.DMA,
            pltpu.SemaphoreType.DMA,
            pltpu.SemaphoreType.DMA,
            pltpu.SemaphoreType.DMA,
            pltpu.SemaphoreType.DMA,
            pltpu.SemaphoreType.DMA,
            pltpu.SemaphoreType.DMA,
            pltpu.SemaphoreType.DMA,
        ],
    )
    def agg_kernel(yl_hbm, yr_hbm, src_hbm, dst_hbm, zz_hbm, zl_hbm, zr_hbm,
                   srcv, dstv, b0, b1, b2, b3,
                   z_sp, g0, g1, g2, g3, s0, s1, s2, s3):
        cid = lax.axis_index("c")
        sid = lax.axis_index("s")
        zsl = pl.ds(sid * ZROWS_PT, ZROWS_PT)
        bufs = (b0, b1, b2, b3)
        gsems = (g0, g1, g2, g3)
        ssems = (s0, s1, s2, s3)
        pltpu.sync_copy(zz_hbm, z_sp.at[zsl])
        plsc.subcore_barrier()

        def run(y_hbm):
            def gather(c, j):
                pltpu.async_copy(y_hbm.at[srcv.at[c]], bufs[j], gsems[j])

            def gwait(j):
                pltpu.make_async_copy(y_hbm.at[srcv.at[0]], bufs[j],
                                      gsems[j]).wait()

            def scat(c, j):
                pltpu.async_copy(bufs[j], z_sp.at[dstv.at[c]], ssems[j],
                                 add=True)

            def swait(j):
                pltpu.make_async_copy(bufs[j], z_sp.at[dstv.at[0]],
                                      ssems[j]).wait()

            def load_qtr(h):
                pltpu.sync_copy(src_hbm.at[sid, pl.ds(h * QTR_A, QTR_A)],
                                srcv)
                pltpu.sync_copy(dst_hbm.at[sid, pl.ds(h * QTR_A, QTR_A)],
                                dstv)

            load_qtr(0)
            for h in range(4):
                for j in range(4):
                    gather(j, j)

                def body(i, carry):
                    for j in range(4):
                        gwait(j)
                        scat(4 * i + j, j)
                    for j in range(4):
                        swait(j)
                        gather(4 * i + 4 + j, j)
                    return carry

                lax.fori_loop(0, QTR_A // 4 - 1, body, 0)
                for j in range(4):
                    gwait(j)
                    scat(QTR_A - 4 + j, j)
                for j in range(4):
                    swait(j)
                if h < 3:
                    load_qtr(h + 1)

        @pl.when(cid == 0)
        def _():
            run(yl_hbm)

        @pl.when(cid == 1)
        def _():
            run(yr_hbm)

        plsc.subcore_barrier()

        @pl.when(cid == 0)
        def _():
            pltpu.sync_copy(z_sp.at[zsl], zl_hbm.at[zsl])

        @pl.when(cid == 1)
        def _():
            pltpu.sync_copy(z_sp.at[zsl], zr_hbm.at[zsl])

    return agg_kernel(yl, yr, src16, dst16, zeros_z)


def _dis(d0_ref, d1_ref):
    deg = d0_ref[:, 0:1] + d1_ref[:, 0:1] + 1.0
    return lax.rsqrt(deg)


def _scaled_matmul(x_p, W, d0, d1):
    def body(x_ref, w_ref, d0_ref, d1_ref, yl_ref, yr_ref):
        y = jnp.dot(x_ref[...], w_ref[...],
                    preferred_element_type=jnp.float32) * _dis(d0_ref, d1_ref)
        yl_ref[...] = y[:, :HF]
        yr_ref[...] = y[:, HF:]

    return pl.pallas_call(
        body,
        grid=(GRID,),
        in_specs=[
            pl.BlockSpec((RB, F), lambda i: (i, 0)),
            pl.BlockSpec((F, F), lambda i: (0, 0)),
            pl.BlockSpec((RB, 16), lambda i: (i, 0)),
            pl.BlockSpec((RB, 16), lambda i: (i, 0)),
        ],
        out_specs=[pl.BlockSpec((RB, HF), lambda i: (i, 0)),
                   pl.BlockSpec((RB, HF), lambda i: (i, 0))],
        out_shape=[jax.ShapeDtypeStruct((NP, HF), jnp.float32),
                   jax.ShapeDtypeStruct((NP, HF), jnp.float32)],
    )(x_p, W, d0, d1)


def _epilogue_stats(zl, zr, yl, yr, d0, d1, b):
    def body(zl_ref, zr_ref, yl_ref, yr_ref, d0_ref, d1_ref, b_ref,
             out_ref, st_ref, acc):
        i = pl.program_id(0)

        @pl.when(i == 0)
        def _():
            acc[...] = jnp.zeros_like(acc)

        dis = _dis(d0_ref, d1_ref)
        z = jnp.concatenate([zl_ref[...], zr_ref[...]], axis=1)
        y = jnp.concatenate([yl_ref[...], yr_ref[...]], axis=1)
        out = dis * (z + y) + b_ref[...]
        out_ref[...] = out
        rows = i * RB + lax.broadcasted_iota(jnp.int32, (RB, 1), 0)
        m = rows < N
        acc[0:1, :] += jnp.sum(jnp.where(m, out, 0.0), axis=0, keepdims=True)
        acc[1:2, :] += jnp.sum(jnp.where(m, out * out, 0.0), axis=0,
                               keepdims=True)

        @pl.when(i == GRID - 1)
        def _():
            st_ref[...] = acc[...]

    return pl.pallas_call(
        body,
        grid=(GRID,),
        in_specs=[
            pl.BlockSpec((RB, HF), lambda i: (i, 0)),
            pl.BlockSpec((RB, HF), lambda i: (i, 0)),
            pl.BlockSpec((RB, HF), lambda i: (i, 0)),
            pl.BlockSpec((RB, HF), lambda i: (i, 0)),
            pl.BlockSpec((RB, 16), lambda i: (i, 0)),
            pl.BlockSpec((RB, 16), lambda i: (i, 0)),
            pl.BlockSpec((1, F), lambda i: (0, 0)),
        ],
        out_specs=[pl.BlockSpec((RB, F), lambda i: (i, 0)),
                   pl.BlockSpec((2, F), lambda i: (0, 0))],
        out_shape=[jax.ShapeDtypeStruct((NP, F), jnp.float32),
                   jax.ShapeDtypeStruct((2, F), jnp.float32)],
        scratch_shapes=[pltpu.VMEM((2, F), jnp.float32)],
    )(zl, zr, yl, yr, d0, d1, b)


def _normalize_block(o_ref, st_ref, g_ref, be_ref, a_ref):
    mu = st_ref[0:1, :] * (1.0 / N)
    var = st_ref[1:2, :] * (1.0 / N) - mu * mu
    inv = lax.rsqrt(var + 1e-5)
    o = (o_ref[...] - mu) * (inv * g_ref[...]) + be_ref[...]
    o = jnp.where(o >= 0, o, a_ref[...] * o)
    nrm = jnp.sqrt(jnp.sum(o * o, axis=1, keepdims=True))
    return o / jnp.maximum(nrm, 1e-12)


def _bn_matmul(out1, st, g, be, a, W2, d0, d1):
    def body(o_ref, st_ref, g_ref, be_ref, a_ref, w_ref, d0_ref, d1_ref,
             yl_ref, yr_ref):
        h = _normalize_block(o_ref, st_ref, g_ref, be_ref, a_ref)
        y2 = jnp.dot(h, w_ref[...],
                     preferred_element_type=jnp.float32) * _dis(d0_ref, d1_ref)
        rows = pl.program_id(0) * RB + lax.broadcasted_iota(
            jnp.int32, (RB, 1), 0)
        y2 = jnp.where(rows < N, y2, 0.0)
        yl_ref[...] = y2[:, :HF]
        yr_ref[...] = y2[:, HF:]

    return pl.pallas_call(
        body,
        grid=(GRID,),
        in_specs=[
            pl.BlockSpec((RB, F), lambda i: (i, 0)),
            pl.BlockSpec((2, F), lambda i: (0, 0)),
            pl.BlockSpec((1, F), lambda i: (0, 0)),
            pl.BlockSpec((1, F), lambda i: (0, 0)),
            pl.BlockSpec((1, 1), lambda i: (0, 0)),
            pl.BlockSpec((F, F), lambda i: (0, 0)),
            pl.BlockSpec((RB, 16), lambda i: (i, 0)),
            pl.BlockSpec((RB, 16), lambda i: (i, 0)),
        ],
        out_specs=[pl.BlockSpec((RB, HF), lambda i: (i, 0)),
                   pl.BlockSpec((RB, HF), lambda i: (i, 0))],
        out_shape=[jax.ShapeDtypeStruct((NP, HF), jnp.float32),
                   jax.ShapeDtypeStruct((NP, HF), jnp.float32)],
    )(out1, st, g, be, a, W2, d0, d1)


def _final(out2, st, g, be, a, batch_b, Wf1, bf1, Wo, bo):
    def body(o_ref, st_ref, g_ref, be_ref, a_ref, bt_ref, wf_ref, bf_ref,
             wo_ref, bo_ref, out_ref, psum, cnt):
        i = pl.program_id(0)

        @pl.when(i == 0)
        def _():
            psum[...] = jnp.zeros_like(psum)
            cnt[...] = jnp.zeros_like(cnt)

        h = _normalize_block(o_ref, st_ref, g_ref, be_ref, a_ref)
        oh = (bt_ref[...] == lax.broadcasted_iota(jnp.int32, (RB, 128), 1)
              ).astype(jnp.float32)
        dn = (((0,), (0,)), ((), ()))
        psum[...] += lax.dot_general(oh, h, dn,
                                     preferred_element_type=jnp.float32)
        cnt[...] += lax.dot_general(oh, jnp.ones((RB, F), jnp.float32), dn,
                                    preferred_element_type=jnp.float32)

        @pl.when(i == GRID - 1)
        def _():
            pooled = psum[...] / jnp.maximum(cnt[...], 1.0)
            p = pooled[0:G, :]
            fc = jnp.maximum(
                jnp.dot(p, wf_ref[...], preferred_element_type=jnp.float32)
                + bf_ref[...], 0.0)
            out_ref[...] = jnp.dot(fc, wo_ref[...],
                                   preferred_element_type=jnp.float32) \
                + bo_ref[...]

    return pl.pallas_call(
        body,
        grid=(GRID,),
        in_specs=[
            pl.BlockSpec((RB, F), lambda i: (i, 0)),
            pl.BlockSpec((2, F), lambda i: (0, 0)),
            pl.BlockSpec((1, F), lambda i: (0, 0)),
            pl.BlockSpec((1, F), lambda i: (0, 0)),
            pl.BlockSpec((1, 1), lambda i: (0, 0)),
            pl.BlockSpec((RB, 128), lambda i: (i, 0)),
            pl.BlockSpec((F, FC1), lambda i: (0, 0)),
            pl.BlockSpec((1, FC1), lambda i: (0, 0)),
            pl.BlockSpec((FC1, 1), lambda i: (0, 0)),
            pl.BlockSpec((1, 1), lambda i: (0, 0)),
        ],
        out_specs=pl.BlockSpec((G, 1), lambda i: (0, 0)),
        out_shape=jax.ShapeDtypeStruct((G, 1), jnp.float32),
        scratch_shapes=[pltpu.VMEM((128, F), jnp.float32),
                        pltpu.VMEM((128, F), jnp.float32)],
    )(out2, st, g, be, a, batch_b, Wf1, bf1, Wo, bo)


def kernel(x, edge_index, batch, W1, b1, g1, be1, a1, W2, b2, g2, be2, a2,
           Wf1, bf1, Wo, bo):
    f32 = jnp.float32
    x_p = jnp.pad(x, ((0, NP - N), (0, 0)))
    pad_e = jnp.full((EP - E,), DUMMY, jnp.int32)
    src = jnp.concatenate([edge_index[0], pad_e])
    dst_deg = jnp.concatenate([edge_index[1], pad_e])
    dst_agg = jnp.concatenate([edge_index[1], jnp.full((EP - E,), N, jnp.int32)])
    src16 = src.reshape(16, NCH_AGG, CHA)
    dst16 = dst_agg.reshape(16, NCH_AGG, CHA)
    dst32 = dst_deg.reshape(32, NCH_DEG, CH)
    batch_p = jnp.pad(batch.astype(jnp.int32), (0, NP - N), constant_values=G)
    batch_b = jnp.broadcast_to(batch_p[:, None], (NP, 128))
    zeros_z = jnp.zeros((ZROWS_PT, HF), f32)
    zeros_d = jnp.zeros((ROWS_PT, 16), f32)
    e0rows = jnp.concatenate(
        [jnp.ones((CH, 1), f32), jnp.zeros((CH, 15), f32)], axis=1)

    b1r = b1.reshape(1, F)
    b2r = b2.reshape(1, F)
    g1r = g1.reshape(1, F)
    be1r = be1.reshape(1, F)
    g2r = g2.reshape(1, F)
    be2r = be2.reshape(1, F)
    a1r = a1.reshape(1, 1)
    a2r = a2.reshape(1, 1)
    bf1r = bf1.reshape(1, FC1)
    bor = bo.reshape(1, 1)

    d0, d1 = _sc_degree(dst32, e0rows, zeros_d)
    yl1, yr1 = _scaled_matmul(x_p, W1, d0, d1)
    zl1, zr1 = _sc_aggregate(yl1, yr1, src16, dst16, zeros_z)
    out1, st1 = _epilogue_stats(zl1, zr1, yl1, yr1, d0, d1, b1r)
    yl2, yr2 = _bn_matmul(out1, st1, g1r, be1r, a1r, W2, d0, d1)
    zl2, zr2 = _sc_aggregate(yl2, yr2, src16, dst16, zeros_z)
    out2, st2 = _epilogue_stats(zl2, zr2, yl2, yr2, d0, d1, b2r)
    return _final(out2, st2, g2r, be2r, a2r, batch_b, Wf1, bf1r, Wo, bor)


# speedup vs baseline: 1.1208x; 1.1208x over previous
import functools

import jax
import jax.numpy as jnp
from jax import lax
from jax.experimental import pallas as pl
from jax.experimental.pallas import tpu as pltpu
from jax.experimental.pallas import tpu_sc as plsc

N = 10000
NP = 10240
F = 256
HF = 128
FC1 = 128
E = 160000
EP = 163840
G = 64
DUMMY = 10200
RB = 1024
GRID = NP // RB
CH = 128
CHA = 128
NCH_AGG = (EP // 16) // CHA
SLC_A = 16
NCH_DEG = (EP // 32) // CH
ROWS_PT = NP // 16
NZ = 10112
ZROWS_PT = NZ // 16


def _sc_degree(dst32, e0rows, zeros_d):
    mesh = plsc.VectorSubcoreMesh(core_axis_name="c", subcore_axis_name="s")

    @functools.partial(
        pl.kernel,
        out_type=(jax.ShapeDtypeStruct((NP, 16), jnp.float32),
                  jax.ShapeDtypeStruct((NP, 16), jnp.float32)),
        mesh=mesh,
        scratch_types=[
            pltpu.VMEM((NCH_DEG, CH), jnp.int32),
            pltpu.VMEM((CH, 16), jnp.float32),
            pltpu.VMEM_SHARED((NP, 16), jnp.float32),
        ],
    )
    def deg_kernel(dst_hbm, e0_hbm, zd_hbm, d0_hbm, d1_hbm, dstv, e0v, deg_sp):
        cid = lax.axis_index("c")
        sid = lax.axis_index("s")
        w = cid * 16 + sid
        sl = pl.ds(sid * ROWS_PT, ROWS_PT)
        pltpu.sync_copy(zd_hbm, deg_sp.at[sl])
        pltpu.sync_copy(e0_hbm, e0v)
        pltpu.sync_copy(dst_hbm.at[w], dstv)
        plsc.subcore_barrier()

        def body(i, carry):
            pltpu.sync_copy(e0v, deg_sp.at[dstv.at[i]], add=True)
            return carry

        lax.fori_loop(0, NCH_DEG, body, 0)
        plsc.subcore_barrier()

        @pl.when(cid == 0)
        def _():
            pltpu.sync_copy(deg_sp.at[sl], d0_hbm.at[sl])

        @pl.when(cid == 1)
        def _():
            pltpu.sync_copy(deg_sp.at[sl], d1_hbm.at[sl])

    return deg_kernel(dst32, e0rows, zeros_d)


def _sc_aggregate(yl, yr, src16, dst16, zeros_z):
    mesh = plsc.VectorSubcoreMesh(core_axis_name="c", subcore_axis_name="s")

    @functools.partial(
        pl.kernel,
        out_type=(jax.ShapeDtypeStruct((NP, HF), jnp.float32),
                  jax.ShapeDtypeStruct((NP, HF), jnp.float32)),
        mesh=mesh,
        scratch_types=[
            pltpu.VMEM((SLC_A, CHA), jnp.int32),
            pltpu.VMEM((SLC_A, CHA), jnp.int32),
            pltpu.VMEM((CHA, HF), jnp.float32),
            pltpu.VMEM((CHA, HF), jnp.float32),
            pltpu.VMEM_SHARED((NZ, HF), jnp.float32),
            pltpu.SemaphoreType.DMA,
            pltpu.SemaphoreType.DMA,
        ],
    )
    def agg_kernel(yl_hbm, yr_hbm, src_hbm, dst_hbm, zz_hbm, zl_hbm, zr_hbm,
                   srcv, dstv, b0, b1, z_sp, g0, g1):
        cid = lax.axis_index("c")
        sid = lax.axis_index("s")
        zsl = pl.ds(sid * ZROWS_PT, ZROWS_PT)
        bufs = (b0, b1)
        gsems = (g0, g1)
        pltpu.sync_copy(zz_hbm, z_sp.at[zsl])
        plsc.subcore_barrier()

        def run(y_hbm):
            def gather(c, j):
                pltpu.async_copy(y_hbm.at[srcv.at[c]], bufs[j], gsems[j])

            def gwait(j):
                pltpu.make_async_copy(y_hbm.at[srcv.at[0]], bufs[j],
                                      gsems[j]).wait()

            def scat(c, j):
                pltpu.sync_copy(bufs[j], z_sp.at[dstv.at[c]], add=True)

            def load_slc(h):
                pltpu.sync_copy(src_hbm.at[sid, pl.ds(h * SLC_A, SLC_A)],
                                srcv)
                pltpu.sync_copy(dst_hbm.at[sid, pl.ds(h * SLC_A, SLC_A)],
                                dstv)

            load_slc(0)
            for h in range(NCH_AGG // SLC_A):
                gather(0, 0)
                gather(1, 1)

                def body(i, carry):
                    gwait(0)
                    scat(2 * i, 0)
                    gather(2 * i + 2, 0)
                    gwait(1)
                    scat(2 * i + 1, 1)
                    gather(2 * i + 3, 1)
                    return carry

                lax.fori_loop(0, SLC_A // 2 - 1, body, 0)
                gwait(0)
                scat(SLC_A - 2, 0)
                gwait(1)
                scat(SLC_A - 1, 1)
                if h < NCH_AGG // SLC_A - 1:
                    load_slc(h + 1)

        @pl.when(cid == 0)
        def _():
            run(yl_hbm)

        @pl.when(cid == 1)
        def _():
            run(yr_hbm)

        plsc.subcore_barrier()

        @pl.when(cid == 0)
        def _():
            pltpu.sync_copy(z_sp.at[zsl], zl_hbm.at[zsl])

        @pl.when(cid == 1)
        def _():
            pltpu.sync_copy(z_sp.at[zsl], zr_hbm.at[zsl])

    return agg_kernel(yl, yr, src16, dst16, zeros_z)


def _dis(d0_ref, d1_ref):
    deg = d0_ref[:, 0:1] + d1_ref[:, 0:1] + 1.0
    return lax.rsqrt(deg)


def _scaled_matmul(x_p, W, d0, d1):
    def body(x_ref, w_ref, d0_ref, d1_ref, yl_ref, yr_ref):
        y = jnp.dot(x_ref[...], w_ref[...],
                    preferred_element_type=jnp.float32) * _dis(d0_ref, d1_ref)
        yl_ref[...] = y[:, :HF]
        yr_ref[...] = y[:, HF:]

    return pl.pallas_call(
        body,
        grid=(GRID,),
        in_specs=[
            pl.BlockSpec((RB, F), lambda i: (i, 0)),
            pl.BlockSpec((F, F), lambda i: (0, 0)),
            pl.BlockSpec((RB, 16), lambda i: (i, 0)),
            pl.BlockSpec((RB, 16), lambda i: (i, 0)),
        ],
        out_specs=[pl.BlockSpec((RB, HF), lambda i: (i, 0)),
                   pl.BlockSpec((RB, HF), lambda i: (i, 0))],
        out_shape=[jax.ShapeDtypeStruct((NP, HF), jnp.float32),
                   jax.ShapeDtypeStruct((NP, HF), jnp.float32)],
    )(x_p, W, d0, d1)


def _epilogue_stats(zl, zr, yl, yr, d0, d1, b):
    def body(zl_ref, zr_ref, yl_ref, yr_ref, d0_ref, d1_ref, b_ref,
             out_ref, st_ref, acc):
        i = pl.program_id(0)

        @pl.when(i == 0)
        def _():
            acc[...] = jnp.zeros_like(acc)

        dis = _dis(d0_ref, d1_ref)
        z = jnp.concatenate([zl_ref[...], zr_ref[...]], axis=1)
        y = jnp.concatenate([yl_ref[...], yr_ref[...]], axis=1)
        out = dis * (z + y) + b_ref[...]
        out_ref[...] = out
        rows = i * RB + lax.broadcasted_iota(jnp.int32, (RB, 1), 0)
        m = rows < N
        acc[0:1, :] += jnp.sum(jnp.where(m, out, 0.0), axis=0, keepdims=True)
        acc[1:2, :] += jnp.sum(jnp.where(m, out * out, 0.0), axis=0,
                               keepdims=True)

        @pl.when(i == GRID - 1)
        def _():
            st_ref[...] = acc[...]

    return pl.pallas_call(
        body,
        grid=(GRID,),
        in_specs=[
            pl.BlockSpec((RB, HF), lambda i: (i, 0)),
            pl.BlockSpec((RB, HF), lambda i: (i, 0)),
            pl.BlockSpec((RB, HF), lambda i: (i, 0)),
            pl.BlockSpec((RB, HF), lambda i: (i, 0)),
            pl.BlockSpec((RB, 16), lambda i: (i, 0)),
            pl.BlockSpec((RB, 16), lambda i: (i, 0)),
            pl.BlockSpec((1, F), lambda i: (0, 0)),
        ],
        out_specs=[pl.BlockSpec((RB, F), lambda i: (i, 0)),
                   pl.BlockSpec((2, F), lambda i: (0, 0))],
        out_shape=[jax.ShapeDtypeStruct((NP, F), jnp.float32),
                   jax.ShapeDtypeStruct((2, F), jnp.float32)],
        scratch_shapes=[pltpu.VMEM((2, F), jnp.float32)],
    )(zl, zr, yl, yr, d0, d1, b)


def _normalize_block(o_ref, st_ref, g_ref, be_ref, a_ref):
    mu = st_ref[0:1, :] * (1.0 / N)
    var = st_ref[1:2, :] * (1.0 / N) - mu * mu
    inv = lax.rsqrt(var + 1e-5)
    o = (o_ref[...] - mu) * (inv * g_ref[...]) + be_ref[...]
    o = jnp.where(o >= 0, o, a_ref[...] * o)
    nrm = jnp.sqrt(jnp.sum(o * o, axis=1, keepdims=True))
    return o / jnp.maximum(nrm, 1e-12)


def _bn_matmul(out1, st, g, be, a, W2, d0, d1):
    def body(o_ref, st_ref, g_ref, be_ref, a_ref, w_ref, d0_ref, d1_ref,
             yl_ref, yr_ref):
        h = _normalize_block(o_ref, st_ref, g_ref, be_ref, a_ref)
        y2 = jnp.dot(h, w_ref[...],
                     preferred_element_type=jnp.float32) * _dis(d0_ref, d1_ref)
        rows = pl.program_id(0) * RB + lax.broadcasted_iota(
            jnp.int32, (RB, 1), 0)
        y2 = jnp.where(rows < N, y2, 0.0)
        yl_ref[...] = y2[:, :HF]
        yr_ref[...] = y2[:, HF:]

    return pl.pallas_call(
        body,
        grid=(GRID,),
        in_specs=[
            pl.BlockSpec((RB, F), lambda i: (i, 0)),
            pl.BlockSpec((2, F), lambda i: (0, 0)),
            pl.BlockSpec((1, F), lambda i: (0, 0)),
            pl.BlockSpec((1, F), lambda i: (0, 0)),
            pl.BlockSpec((1, 1), lambda i: (0, 0)),
            pl.BlockSpec((F, F), lambda i: (0, 0)),
            pl.BlockSpec((RB, 16), lambda i: (i, 0)),
            pl.BlockSpec((RB, 16), lambda i: (i, 0)),
        ],
        out_specs=[pl.BlockSpec((RB, HF), lambda i: (i, 0)),
                   pl.BlockSpec((RB, HF), lambda i: (i, 0))],
        out_shape=[jax.ShapeDtypeStruct((NP, HF), jnp.float32),
                   jax.ShapeDtypeStruct((NP, HF), jnp.float32)],
    )(out1, st, g, be, a, W2, d0, d1)


def _final(out2, st, g, be, a, batch_b, Wf1, bf1, Wo, bo):
    def body(o_ref, st_ref, g_ref, be_ref, a_ref, bt_ref, wf_ref, bf_ref,
             wo_ref, bo_ref, out_ref, psum, cnt):
        i = pl.program_id(0)

        @pl.when(i == 0)
        def _():
            psum[...] = jnp.zeros_like(psum)
            cnt[...] = jnp.zeros_like(cnt)

        h = _normalize_block(o_ref, st_ref, g_ref, be_ref, a_ref)
        oh = (bt_ref[...] == lax.broadcasted_iota(jnp.int32, (RB, 128), 1)
              ).astype(jnp.float32)
        dn = (((0,), (0,)), ((), ()))
        psum[...] += lax.dot_general(oh, h, dn,
                                     preferred_element_type=jnp.float32)
        cnt[...] += lax.dot_general(oh, jnp.ones((RB, F), jnp.float32), dn,
                                    preferred_element_type=jnp.float32)

        @pl.when(i == GRID - 1)
        def _():
            pooled = psum[...] / jnp.maximum(cnt[...], 1.0)
            p = pooled[0:G, :]
            fc = jnp.maximum(
                jnp.dot(p, wf_ref[...], preferred_element_type=jnp.float32)
                + bf_ref[...], 0.0)
            out_ref[...] = jnp.dot(fc, wo_ref[...],
                                   preferred_element_type=jnp.float32) \
                + bo_ref[...]

    return pl.pallas_call(
        body,
        grid=(GRID,),
        in_specs=[
            pl.BlockSpec((RB, F), lambda i: (i, 0)),
            pl.BlockSpec((2, F), lambda i: (0, 0)),
            pl.BlockSpec((1, F), lambda i: (0, 0)),
            pl.BlockSpec((1, F), lambda i: (0, 0)),
            pl.BlockSpec((1, 1), lambda i: (0, 0)),
            pl.BlockSpec((RB, 128), lambda i: (i, 0)),
            pl.BlockSpec((F, FC1), lambda i: (0, 0)),
            pl.BlockSpec((1, FC1), lambda i: (0, 0)),
            pl.BlockSpec((FC1, 1), lambda i: (0, 0)),
            pl.BlockSpec((1, 1), lambda i: (0, 0)),
        ],
        out_specs=pl.BlockSpec((G, 1), lambda i: (0, 0)),
        out_shape=jax.ShapeDtypeStruct((G, 1), jnp.float32),
        scratch_shapes=[pltpu.VMEM((128, F), jnp.float32),
                        pltpu.VMEM((128, F), jnp.float32)],
    )(out2, st, g, be, a, batch_b, Wf1, bf1, Wo, bo)


def kernel(x, edge_index, batch, W1, b1, g1, be1, a1, W2, b2, g2, be2, a2,
           Wf1, bf1, Wo, bo):
    f32 = jnp.float32
    x_p = jnp.pad(x, ((0, NP - N), (0, 0)))
    pad_e = jnp.full((EP - E,), DUMMY, jnp.int32)
    src = jnp.concatenate([edge_index[0], pad_e])
    dst_deg = jnp.concatenate([edge_index[1], pad_e])
    dst_agg = jnp.concatenate([edge_index[1], jnp.full((EP - E,), N, jnp.int32)])
    src16 = src.reshape(16, NCH_AGG, CHA)
    dst16 = dst_agg.reshape(16, NCH_AGG, CHA)
    dst32 = dst_deg.reshape(32, NCH_DEG, CH)
    batch_p = jnp.pad(batch.astype(jnp.int32), (0, NP - N), constant_values=G)
    batch_b = jnp.broadcast_to(batch_p[:, None], (NP, 128))
    zeros_z = jnp.zeros((ZROWS_PT, HF), f32)
    zeros_d = jnp.zeros((ROWS_PT, 16), f32)
    e0rows = jnp.concatenate(
        [jnp.ones((CH, 1), f32), jnp.zeros((CH, 15), f32)], axis=1)

    b1r = b1.reshape(1, F)
    b2r = b2.reshape(1, F)
    g1r = g1.reshape(1, F)
    be1r = be1.reshape(1, F)
    g2r = g2.reshape(1, F)
    be2r = be2.reshape(1, F)
    a1r = a1.reshape(1, 1)
    a2r = a2.reshape(1, 1)
    bf1r = bf1.reshape(1, FC1)
    bor = bo.reshape(1, 1)

    d0, d1 = _sc_degree(dst32, e0rows, zeros_d)
    yl1, yr1 = _scaled_matmul(x_p, W1, d0, d1)
    zl1, zr1 = _sc_aggregate(yl1, yr1, src16, dst16, zeros_z)
    out1, st1 = _epilogue_stats(zl1, zr1, yl1, yr1, d0, d1, b1r)
    yl2, yr2 = _bn_matmul(out1, st1, g1r, be1r, a1r, W2, d0, d1)
    zl2, zr2 = _sc_aggregate(yl2, yr2, src16, dst16, zeros_z)
    out2, st2 = _epilogue_stats(zl2, zr2, yl2, yr2, d0, d1, b2r)
    return _final(out2, st2, g2r, be2r, a2r, batch_b, Wf1, bf1r, Wo, bor)


# speedup vs baseline: 1.1452x; 1.0218x over previous
import functools

import jax
import jax.numpy as jnp
from jax import lax
from jax.experimental import pallas as pl
from jax.experimental.pallas import tpu as pltpu
from jax.experimental.pallas import tpu_sc as plsc

N = 10000
NP = 10240
F = 256
HF = 128
FC1 = 128
E = 160000
EP = 163840
G = 64
DUMMY = 10200
RB = 1024
GRID = NP // RB
CH = 128
CHA = 128
NCH_AGG = (EP // 16) // CHA
SLC_A = 40
NCH_DEG = (EP // 32) // CH
ROWS_PT = NP // 16
NZ = 10112
ZROWS_PT = NZ // 16


def _sc_degree(dst32, e0rows, zeros_d):
    mesh = plsc.VectorSubcoreMesh(core_axis_name="c", subcore_axis_name="s")

    @functools.partial(
        pl.kernel,
        out_type=(jax.ShapeDtypeStruct((NP, 16), jnp.float32),
                  jax.ShapeDtypeStruct((NP, 16), jnp.float32)),
        mesh=mesh,
        scratch_types=[
            pltpu.VMEM((NCH_DEG, CH), jnp.int32),
            pltpu.VMEM((CH, 16), jnp.float32),
            pltpu.VMEM_SHARED((NP, 16), jnp.float32),
        ],
    )
    def deg_kernel(dst_hbm, e0_hbm, zd_hbm, d0_hbm, d1_hbm, dstv, e0v, deg_sp):
        cid = lax.axis_index("c")
        sid = lax.axis_index("s")
        w = cid * 16 + sid
        sl = pl.ds(sid * ROWS_PT, ROWS_PT)
        pltpu.sync_copy(zd_hbm, deg_sp.at[sl])
        pltpu.sync_copy(e0_hbm, e0v)
        pltpu.sync_copy(dst_hbm.at[w], dstv)
        plsc.subcore_barrier()

        def body(i, carry):
            pltpu.sync_copy(e0v, deg_sp.at[dstv.at[i]], add=True)
            return carry

        lax.fori_loop(0, NCH_DEG, body, 0)
        plsc.subcore_barrier()

        @pl.when(cid == 0)
        def _():
            pltpu.sync_copy(deg_sp.at[sl], d0_hbm.at[sl])

        @pl.when(cid == 1)
        def _():
            pltpu.sync_copy(deg_sp.at[sl], d1_hbm.at[sl])

    return deg_kernel(dst32, e0rows, zeros_d)


def _sc_aggregate(yl, yr, src16, dst16, zeros_z):
    mesh = plsc.VectorSubcoreMesh(core_axis_name="c", subcore_axis_name="s")

    @functools.partial(
        pl.kernel,
        out_type=(jax.ShapeDtypeStruct((NP, HF), jnp.float32),
                  jax.ShapeDtypeStruct((NP, HF), jnp.float32)),
        mesh=mesh,
        scratch_types=[
            pltpu.VMEM((SLC_A, CHA), jnp.int32),
            pltpu.VMEM((SLC_A, CHA), jnp.int32),
            pltpu.VMEM((CHA, HF), jnp.float32),
            pltpu.VMEM((CHA, HF), jnp.float32),
            pltpu.VMEM_SHARED((NZ, HF), jnp.float32),
            pltpu.SemaphoreType.DMA,
            pltpu.SemaphoreType.DMA,
        ],
    )
    def agg_kernel(yl_hbm, yr_hbm, src_hbm, dst_hbm, zz_hbm, zl_hbm, zr_hbm,
                   srcv, dstv, b0, b1, z_sp, g0, g1):
        cid = lax.axis_index("c")
        sid = lax.axis_index("s")
        zsl = pl.ds(sid * ZROWS_PT, ZROWS_PT)
        bufs = (b0, b1)
        gsems = (g0, g1)
        pltpu.sync_copy(zz_hbm, z_sp.at[zsl])
        plsc.subcore_barrier()

        def run(y_hbm):
            def gather(c, j):
                pltpu.async_copy(y_hbm.at[srcv.at[c]], bufs[j], gsems[j])

            def gwait(j):
                pltpu.make_async_copy(y_hbm.at[srcv.at[0]], bufs[j],
                                      gsems[j]).wait()

            def scat(c, j):
                pltpu.sync_copy(bufs[j], z_sp.at[dstv.at[c]], add=True)

            def load_slc(h):
                pltpu.sync_copy(src_hbm.at[sid, pl.ds(h * SLC_A, SLC_A)],
                                srcv)
                pltpu.sync_copy(dst_hbm.at[sid, pl.ds(h * SLC_A, SLC_A)],
                                dstv)

            load_slc(0)
            for h in range(NCH_AGG // SLC_A):
                gather(0, 0)
                gather(1, 1)

                def body(i, carry):
                    gwait(0)
                    scat(2 * i, 0)
                    gather(2 * i + 2, 0)
                    gwait(1)
                    scat(2 * i + 1, 1)
                    gather(2 * i + 3, 1)
                    return carry

                lax.fori_loop(0, SLC_A // 2 - 1, body, 0)
                gwait(0)
                scat(SLC_A - 2, 0)
                gwait(1)
                scat(SLC_A - 1, 1)
                if h < NCH_AGG // SLC_A - 1:
                    load_slc(h + 1)

        @pl.when(cid == 0)
        def _():
            run(yl_hbm)

        @pl.when(cid == 1)
        def _():
            run(yr_hbm)

        plsc.subcore_barrier()

        @pl.when(cid == 0)
        def _():
            pltpu.sync_copy(z_sp.at[zsl], zl_hbm.at[zsl])

        @pl.when(cid == 1)
        def _():
            pltpu.sync_copy(z_sp.at[zsl], zr_hbm.at[zsl])

    return agg_kernel(yl, yr, src16, dst16, zeros_z)


def _dis(d0_ref, d1_ref):
    deg = d0_ref[:, 0:1] + d1_ref[:, 0:1] + 1.0
    return lax.rsqrt(deg)


def _scaled_matmul(x_p, W, d0, d1):
    def body(x_ref, w_ref, d0_ref, d1_ref, yl_ref, yr_ref):
        y = jnp.dot(x_ref[...], w_ref[...],
                    preferred_element_type=jnp.float32) * _dis(d0_ref, d1_ref)
        yl_ref[...] = y[:, :HF]
        yr_ref[...] = y[:, HF:]

    return pl.pallas_call(
        body,
        grid=(GRID,),
        in_specs=[
            pl.BlockSpec((RB, F), lambda i: (i, 0)),
            pl.BlockSpec((F, F), lambda i: (0, 0)),
            pl.BlockSpec((RB, 16), lambda i: (i, 0)),
            pl.BlockSpec((RB, 16), lambda i: (i, 0)),
        ],
        out_specs=[pl.BlockSpec((RB, HF), lambda i: (i, 0)),
                   pl.BlockSpec((RB, HF), lambda i: (i, 0))],
        out_shape=[jax.ShapeDtypeStruct((NP, HF), jnp.float32),
                   jax.ShapeDtypeStruct((NP, HF), jnp.float32)],
    )(x_p, W, d0, d1)


def _epilogue_stats(zl, zr, yl, yr, d0, d1, b):
    def body(zl_ref, zr_ref, yl_ref, yr_ref, d0_ref, d1_ref, b_ref,
             out_ref, st_ref, acc):
        i = pl.program_id(0)

        @pl.when(i == 0)
        def _():
            acc[...] = jnp.zeros_like(acc)

        dis = _dis(d0_ref, d1_ref)
        z = jnp.concatenate([zl_ref[...], zr_ref[...]], axis=1)
        y = jnp.concatenate([yl_ref[...], yr_ref[...]], axis=1)
        out = dis * (z + y) + b_ref[...]
        out_ref[...] = out
        rows = i * RB + lax.broadcasted_iota(jnp.int32, (RB, 1), 0)
        m = rows < N
        acc[0:1, :] += jnp.sum(jnp.where(m, out, 0.0), axis=0, keepdims=True)
        acc[1:2, :] += jnp.sum(jnp.where(m, out * out, 0.0), axis=0,
                               keepdims=True)

        @pl.when(i == GRID - 1)
        def _():
            st_ref[...] = acc[...]

    return pl.pallas_call(
        body,
        grid=(GRID,),
        in_specs=[
            pl.BlockSpec((RB, HF), lambda i: (i, 0)),
            pl.BlockSpec((RB, HF), lambda i: (i, 0)),
            pl.BlockSpec((RB, HF), lambda i: (i, 0)),
            pl.BlockSpec((RB, HF), lambda i: (i, 0)),
            pl.BlockSpec((RB, 16), lambda i: (i, 0)),
            pl.BlockSpec((RB, 16), lambda i: (i, 0)),
            pl.BlockSpec((1, F), lambda i: (0, 0)),
        ],
        out_specs=[pl.BlockSpec((RB, F), lambda i: (i, 0)),
                   pl.BlockSpec((2, F), lambda i: (0, 0))],
        out_shape=[jax.ShapeDtypeStruct((NP, F), jnp.float32),
                   jax.ShapeDtypeStruct((2, F), jnp.float32)],
        scratch_shapes=[pltpu.VMEM((2, F), jnp.float32)],
    )(zl, zr, yl, yr, d0, d1, b)


def _normalize_block(o_ref, st_ref, g_ref, be_ref, a_ref):
    mu = st_ref[0:1, :] * (1.0 / N)
    var = st_ref[1:2, :] * (1.0 / N) - mu * mu
    inv = lax.rsqrt(var + 1e-5)
    o = (o_ref[...] - mu) * (inv * g_ref[...]) + be_ref[...]
    o = jnp.where(o >= 0, o, a_ref[...] * o)
    nrm = jnp.sqrt(jnp.sum(o * o, axis=1, keepdims=True))
    return o / jnp.maximum(nrm, 1e-12)


def _bn_matmul(out1, st, g, be, a, W2, d0, d1):
    def body(o_ref, st_ref, g_ref, be_ref, a_ref, w_ref, d0_ref, d1_ref,
             yl_ref, yr_ref):
        h = _normalize_block(o_ref, st_ref, g_ref, be_ref, a_ref)
        y2 = jnp.dot(h, w_ref[...],
                     preferred_element_type=jnp.float32) * _dis(d0_ref, d1_ref)
        rows = pl.program_id(0) * RB + lax.broadcasted_iota(
            jnp.int32, (RB, 1), 0)
        y2 = jnp.where(rows < N, y2, 0.0)
        yl_ref[...] = y2[:, :HF]
        yr_ref[...] = y2[:, HF:]

    return pl.pallas_call(
        body,
        grid=(GRID,),
        in_specs=[
            pl.BlockSpec((RB, F), lambda i: (i, 0)),
            pl.BlockSpec((2, F), lambda i: (0, 0)),
            pl.BlockSpec((1, F), lambda i: (0, 0)),
            pl.BlockSpec((1, F), lambda i: (0, 0)),
            pl.BlockSpec((1, 1), lambda i: (0, 0)),
            pl.BlockSpec((F, F), lambda i: (0, 0)),
            pl.BlockSpec((RB, 16), lambda i: (i, 0)),
            pl.BlockSpec((RB, 16), lambda i: (i, 0)),
        ],
        out_specs=[pl.BlockSpec((RB, HF), lambda i: (i, 0)),
                   pl.BlockSpec((RB, HF), lambda i: (i, 0))],
        out_shape=[jax.ShapeDtypeStruct((NP, HF), jnp.float32),
                   jax.ShapeDtypeStruct((NP, HF), jnp.float32)],
    )(out1, st, g, be, a, W2, d0, d1)


def _final(out2, st, g, be, a, batch_b, Wf1, bf1, Wo, bo):
    def body(o_ref, st_ref, g_ref, be_ref, a_ref, bt_ref, wf_ref, bf_ref,
             wo_ref, bo_ref, out_ref, psum, cnt):
        i = pl.program_id(0)

        @pl.when(i == 0)
        def _():
            psum[...] = jnp.zeros_like(psum)
            cnt[...] = jnp.zeros_like(cnt)

        h = _normalize_block(o_ref, st_ref, g_ref, be_ref, a_ref)
        oh = (bt_ref[...] == lax.broadcasted_iota(jnp.int32, (RB, 128), 1)
              ).astype(jnp.float32)
        dn = (((0,), (0,)), ((), ()))
        psum[...] += lax.dot_general(oh, h, dn,
                                     preferred_element_type=jnp.float32)
        cnt[...] += lax.dot_general(oh, jnp.ones((RB, F), jnp.float32), dn,
                                    preferred_element_type=jnp.float32)

        @pl.when(i == GRID - 1)
        def _():
            pooled = psum[...] / jnp.maximum(cnt[...], 1.0)
            p = pooled[0:G, :]
            fc = jnp.maximum(
                jnp.dot(p, wf_ref[...], preferred_element_type=jnp.float32)
                + bf_ref[...], 0.0)
            out_ref[...] = jnp.dot(fc, wo_ref[...],
                                   preferred_element_type=jnp.float32) \
                + bo_ref[...]

    return pl.pallas_call(
        body,
        grid=(GRID,),
        in_specs=[
            pl.BlockSpec((RB, F), lambda i: (i, 0)),
            pl.BlockSpec((2, F), lambda i: (0, 0)),
            pl.BlockSpec((1, F), lambda i: (0, 0)),
            pl.BlockSpec((1, F), lambda i: (0, 0)),
            pl.BlockSpec((1, 1), lambda i: (0, 0)),
            pl.BlockSpec((RB, 128), lambda i: (i, 0)),
            pl.BlockSpec((F, FC1), lambda i: (0, 0)),
            pl.BlockSpec((1, FC1), lambda i: (0, 0)),
            pl.BlockSpec((FC1, 1), lambda i: (0, 0)),
            pl.BlockSpec((1, 1), lambda i: (0, 0)),
        ],
        out_specs=pl.BlockSpec((G, 1), lambda i: (0, 0)),
        out_shape=jax.ShapeDtypeStruct((G, 1), jnp.float32),
        scratch_shapes=[pltpu.VMEM((128, F), jnp.float32),
                        pltpu.VMEM((128, F), jnp.float32)],
    )(out2, st, g, be, a, batch_b, Wf1, bf1, Wo, bo)


def kernel(x, edge_index, batch, W1, b1, g1, be1, a1, W2, b2, g2, be2, a2,
           Wf1, bf1, Wo, bo):
    f32 = jnp.float32
    x_p = jnp.pad(x, ((0, NP - N), (0, 0)))
    pad_e = jnp.full((EP - E,), DUMMY, jnp.int32)
    src = jnp.concatenate([edge_index[0], pad_e])
    dst_deg = jnp.concatenate([edge_index[1], pad_e])
    dst_agg = jnp.concatenate([edge_index[1], jnp.full((EP - E,), N, jnp.int32)])
    src16 = src.reshape(16, NCH_AGG, CHA)
    dst16 = dst_agg.reshape(16, NCH_AGG, CHA)
    dst32 = dst_deg.reshape(32, NCH_DEG, CH)
    batch_p = jnp.pad(batch.astype(jnp.int32), (0, NP - N), constant_values=G)
    batch_b = jnp.broadcast_to(batch_p[:, None], (NP, 128))
    zeros_z = jnp.zeros((ZROWS_PT, HF), f32)
    zeros_d = jnp.zeros((ROWS_PT, 16), f32)
    e0rows = jnp.concatenate(
        [jnp.ones((CH, 1), f32), jnp.zeros((CH, 15), f32)], axis=1)

    b1r = b1.reshape(1, F)
    b2r = b2.reshape(1, F)
    g1r = g1.reshape(1, F)
    be1r = be1.reshape(1, F)
    g2r = g2.reshape(1, F)
    be2r = be2.reshape(1, F)
    a1r = a1.reshape(1, 1)
    a2r = a2.reshape(1, 1)
    bf1r = bf1.reshape(1, FC1)
    bor = bo.reshape(1, 1)

    d0, d1 = _sc_degree(dst32, e0rows, zeros_d)
    yl1, yr1 = _scaled_matmul(x_p, W1, d0, d1)
    zl1, zr1 = _sc_aggregate(yl1, yr1, src16, dst16, zeros_z)
    out1, st1 = _epilogue_stats(zl1, zr1, yl1, yr1, d0, d1, b1r)
    yl2, yr2 = _bn_matmul(out1, st1, g1r, be1r, a1r, W2, d0, d1)
    zl2, zr2 = _sc_aggregate(yl2, yr2, src16, dst16, zeros_z)
    out2, st2 = _epilogue_stats(zl2, zr2, yl2, yr2, d0, d1, b2r)
    return _final(out2, st2, g2r, be2r, a2r, batch_b, Wf1, bf1r, Wo, bor)


# speedup vs baseline: 1.1851x; 1.0348x over previous
import functools

import jax
import jax.numpy as jnp
from jax import lax
from jax.experimental import pallas as pl
from jax.experimental.pallas import tpu as pltpu
from jax.experimental.pallas import tpu_sc as plsc

N = 10000
NP = 10240
F = 256
HF = 128
FC1 = 128
E = 160000
EP = 163840
G = 64
DUMMY = 10200
RB = 1024
GRID = NP // RB
CH = 128
CHA = 128
NCH_AGG = (EP // 16) // CHA
SLC_A = 40
NCH_DEG = (EP // 32) // CH
ROWS_PT = NP // 16
NZ = NP
ZROWS_PT = NZ // 16


def _sc_degree(dst32, e0rows, zeros_d):
    mesh = plsc.VectorSubcoreMesh(core_axis_name="c", subcore_axis_name="s")

    @functools.partial(
        pl.kernel,
        out_type=(jax.ShapeDtypeStruct((NP, 16), jnp.float32),
                  jax.ShapeDtypeStruct((NP, 16), jnp.float32)),
        mesh=mesh,
        scratch_types=[
            pltpu.VMEM((NCH_DEG, CH), jnp.int32),
            pltpu.VMEM((CH, 16), jnp.float32),
            pltpu.VMEM_SHARED((NP, 16), jnp.float32),
        ],
    )
    def deg_kernel(dst_hbm, e0_hbm, zd_hbm, d0_hbm, d1_hbm, dstv, e0v, deg_sp):
        cid = lax.axis_index("c")
        sid = lax.axis_index("s")
        w = cid * 16 + sid
        sl = pl.ds(sid * ROWS_PT, ROWS_PT)
        pltpu.sync_copy(zd_hbm, deg_sp.at[sl])
        pltpu.sync_copy(e0_hbm, e0v)
        pltpu.sync_copy(dst_hbm.at[w], dstv)
        plsc.subcore_barrier()

        def body(i, carry):
            pltpu.sync_copy(e0v, deg_sp.at[dstv.at[i]], add=True)
            return carry

        lax.fori_loop(0, NCH_DEG, body, 0)
        plsc.subcore_barrier()

        @pl.when(cid == 0)
        def _():
            pltpu.sync_copy(deg_sp.at[sl], d0_hbm.at[sl])

        @pl.when(cid == 1)
        def _():
            pltpu.sync_copy(deg_sp.at[sl], d1_hbm.at[sl])

    return deg_kernel(dst32, e0rows, zeros_d)


def _sc_aggregate(yl, yr, src16, dst16, zeros_z):
    mesh = plsc.VectorSubcoreMesh(core_axis_name="c", subcore_axis_name="s")

    @functools.partial(
        pl.kernel,
        out_type=(jax.ShapeDtypeStruct((NP, HF), jnp.float32),
                  jax.ShapeDtypeStruct((NP, HF), jnp.float32)),
        mesh=mesh,
        scratch_types=[
            pltpu.VMEM((SLC_A, CHA), jnp.int32),
            pltpu.VMEM((SLC_A, CHA), jnp.int32),
            pltpu.VMEM((CHA, HF), jnp.float32),
            pltpu.VMEM((CHA, HF), jnp.float32),
            pltpu.VMEM_SHARED((NZ, HF), jnp.float32),
            pltpu.SemaphoreType.DMA,
            pltpu.SemaphoreType.DMA,
        ],
    )
    def agg_kernel(yl_hbm, yr_hbm, src_hbm, dst_hbm, zz_hbm, zl_hbm, zr_hbm,
                   srcv, dstv, b0, b1, z_sp, g0, g1):
        cid = lax.axis_index("c")
        sid = lax.axis_index("s")
        zsl = pl.ds(sid * ZROWS_PT, ZROWS_PT)
        bufs = (b0, b1)
        gsems = (g0, g1)
        pltpu.sync_copy(zz_hbm, z_sp.at[zsl])
        plsc.subcore_barrier()

        def run(y_hbm):
            def gather(c, j):
                pltpu.async_copy(y_hbm.at[srcv.at[c]], bufs[j], gsems[j])

            def gwait(j):
                pltpu.make_async_copy(y_hbm.at[srcv.at[0]], bufs[j],
                                      gsems[j]).wait()

            def scat(c, j):
                pltpu.sync_copy(bufs[j], z_sp.at[dstv.at[c]], add=True)

            def load_slc(h):
                pltpu.sync_copy(src_hbm.at[sid, pl.ds(h * SLC_A, SLC_A)],
                                srcv)
                pltpu.sync_copy(dst_hbm.at[sid, pl.ds(h * SLC_A, SLC_A)],
                                dstv)

            load_slc(0)
            for h in range(NCH_AGG // SLC_A):
                gather(0, 0)
                gather(1, 1)

                def body(i, carry):
                    gwait(0)
                    scat(2 * i, 0)
                    gather(2 * i + 2, 0)
                    gwait(1)
                    scat(2 * i + 1, 1)
                    gather(2 * i + 3, 1)
                    return carry

                lax.fori_loop(0, SLC_A // 2 - 1, body, 0)
                gwait(0)
                scat(SLC_A - 2, 0)
                gwait(1)
                scat(SLC_A - 1, 1)
                if h < NCH_AGG // SLC_A - 1:
                    load_slc(h + 1)

        @pl.when(cid == 0)
        def _():
            run(yl_hbm)

        @pl.when(cid == 1)
        def _():
            run(yr_hbm)

        plsc.subcore_barrier()

        @pl.when(cid == 0)
        def _():
            pltpu.sync_copy(z_sp.at[zsl], zl_hbm.at[zsl])

        @pl.when(cid == 1)
        def _():
            pltpu.sync_copy(z_sp.at[zsl], zr_hbm.at[zsl])

    return agg_kernel(yl, yr, src16, dst16, zeros_z)


def _dis(d0_ref, d1_ref):
    deg = d0_ref[:, 0:1] + d1_ref[:, 0:1] + 1.0
    return lax.rsqrt(deg)


def _scaled_matmul(x_p, W, d0, d1):
    def body(x_ref, w_ref, d0_ref, d1_ref, yl_ref, yr_ref):
        y = jnp.dot(x_ref[...], w_ref[...],
                    preferred_element_type=jnp.float32) * _dis(d0_ref, d1_ref)
        yl_ref[...] = y[:, :HF]
        yr_ref[...] = y[:, HF:]

    return pl.pallas_call(
        body,
        grid=(GRID,),
        in_specs=[
            pl.BlockSpec((RB, F), lambda i: (i, 0)),
            pl.BlockSpec((F, F), lambda i: (0, 0)),
            pl.BlockSpec((RB, 16), lambda i: (i, 0)),
            pl.BlockSpec((RB, 16), lambda i: (i, 0)),
        ],
        out_specs=[pl.BlockSpec((RB, HF), lambda i: (i, 0)),
                   pl.BlockSpec((RB, HF), lambda i: (i, 0))],
        out_shape=[jax.ShapeDtypeStruct((NP, HF), jnp.float32),
                   jax.ShapeDtypeStruct((NP, HF), jnp.float32)],
    )(x_p, W, d0, d1)


def _epilogue_stats(zl, zr, yl, yr, d0, d1, b):
    def body(zl_ref, zr_ref, yl_ref, yr_ref, d0_ref, d1_ref, b_ref,
             out_ref, st_ref, acc):
        i = pl.program_id(0)

        @pl.when(i == 0)
        def _():
            acc[...] = jnp.zeros_like(acc)

        dis = _dis(d0_ref, d1_ref)
        z = jnp.concatenate([zl_ref[...], zr_ref[...]], axis=1)
        y = jnp.concatenate([yl_ref[...], yr_ref[...]], axis=1)
        out = dis * (z + y) + b_ref[...]
        out_ref[...] = out
        rows = i * RB + lax.broadcasted_iota(jnp.int32, (RB, 1), 0)
        m = rows < N
        acc[0:1, :] += jnp.sum(jnp.where(m, out, 0.0), axis=0, keepdims=True)
        acc[1:2, :] += jnp.sum(jnp.where(m, out * out, 0.0), axis=0,
                               keepdims=True)

        @pl.when(i == GRID - 1)
        def _():
            st_ref[...] = acc[...]

    return pl.pallas_call(
        body,
        grid=(GRID,),
        in_specs=[
            pl.BlockSpec((RB, HF), lambda i: (i, 0)),
            pl.BlockSpec((RB, HF), lambda i: (i, 0)),
            pl.BlockSpec((RB, HF), lambda i: (i, 0)),
            pl.BlockSpec((RB, HF), lambda i: (i, 0)),
            pl.BlockSpec((RB, 16), lambda i: (i, 0)),
            pl.BlockSpec((RB, 16), lambda i: (i, 0)),
            pl.BlockSpec((1, F), lambda i: (0, 0)),
        ],
        out_specs=[pl.BlockSpec((RB, F), lambda i: (i, 0)),
                   pl.BlockSpec((2, F), lambda i: (0, 0))],
        out_shape=[jax.ShapeDtypeStruct((NP, F), jnp.float32),
                   jax.ShapeDtypeStruct((2, F), jnp.float32)],
        scratch_shapes=[pltpu.VMEM((2, F), jnp.float32)],
    )(zl, zr, yl, yr, d0, d1, b)


def _normalize_block(o_ref, st_ref, g_ref, be_ref, a_ref):
    mu = st_ref[0:1, :] * (1.0 / N)
    var = st_ref[1:2, :] * (1.0 / N) - mu * mu
    inv = lax.rsqrt(var + 1e-5)
    o = (o_ref[...] - mu) * (inv * g_ref[...]) + be_ref[...]
    o = jnp.where(o >= 0, o, a_ref[...] * o)
    nrm = jnp.sqrt(jnp.sum(o * o, axis=1, keepdims=True))
    return o / jnp.maximum(nrm, 1e-12)


def _bn_matmul(out1, st, g, be, a, W2, d0, d1):
    def body(o_ref, st_ref, g_ref, be_ref, a_ref, w_ref, d0_ref, d1_ref,
             yl_ref, yr_ref):
        h = _normalize_block(o_ref, st_ref, g_ref, be_ref, a_ref)
        y2 = jnp.dot(h, w_ref[...],
                     preferred_element_type=jnp.float32) * _dis(d0_ref, d1_ref)
        rows = pl.program_id(0) * RB + lax.broadcasted_iota(
            jnp.int32, (RB, 1), 0)
        y2 = jnp.where(rows < N, y2, 0.0)
        yl_ref[...] = y2[:, :HF]
        yr_ref[...] = y2[:, HF:]

    return pl.pallas_call(
        body,
        grid=(GRID,),
        in_specs=[
            pl.BlockSpec((RB, F), lambda i: (i, 0)),
            pl.BlockSpec((2, F), lambda i: (0, 0)),
            pl.BlockSpec((1, F), lambda i: (0, 0)),
            pl.BlockSpec((1, F), lambda i: (0, 0)),
            pl.BlockSpec((1, 1), lambda i: (0, 0)),
            pl.BlockSpec((F, F), lambda i: (0, 0)),
            pl.BlockSpec((RB, 16), lambda i: (i, 0)),
            pl.BlockSpec((RB, 16), lambda i: (i, 0)),
        ],
        out_specs=[pl.BlockSpec((RB, HF), lambda i: (i, 0)),
                   pl.BlockSpec((RB, HF), lambda i: (i, 0))],
        out_shape=[jax.ShapeDtypeStruct((NP, HF), jnp.float32),
                   jax.ShapeDtypeStruct((NP, HF), jnp.float32)],
    )(out1, st, g, be, a, W2, d0, d1)


def _final(out2, st, g, be, a, batch_b, Wf1, bf1, Wo, bo):
    def body(o_ref, st_ref, g_ref, be_ref, a_ref, bt_ref, wf_ref, bf_ref,
             wo_ref, bo_ref, out_ref, psum, cnt):
        i = pl.program_id(0)

        @pl.when(i == 0)
        def _():
            psum[...] = jnp.zeros_like(psum)
            cnt[...] = jnp.zeros_like(cnt)

        h = _normalize_block(o_ref, st_ref, g_ref, be_ref, a_ref)
        oh = (bt_ref[...] == lax.broadcasted_iota(jnp.int32, (RB, 128), 1)
              ).astype(jnp.float32)
        dn = (((0,), (0,)), ((), ()))
        psum[...] += lax.dot_general(oh, h, dn,
                                     preferred_element_type=jnp.float32)
        cnt[...] += lax.dot_general(oh, jnp.ones((RB, F), jnp.float32), dn,
                                    preferred_element_type=jnp.float32)

        @pl.when(i == GRID - 1)
        def _():
            pooled = psum[...] / jnp.maximum(cnt[...], 1.0)
            p = pooled[0:G, :]
            fc = jnp.maximum(
                jnp.dot(p, wf_ref[...], preferred_element_type=jnp.float32)
                + bf_ref[...], 0.0)
            out_ref[...] = jnp.dot(fc, wo_ref[...],
                                   preferred_element_type=jnp.float32) \
                + bo_ref[...]

    return pl.pallas_call(
        body,
        grid=(GRID,),
        in_specs=[
            pl.BlockSpec((RB, F), lambda i: (i, 0)),
            pl.BlockSpec((2, F), lambda i: (0, 0)),
            pl.BlockSpec((1, F), lambda i: (0, 0)),
            pl.BlockSpec((1, F), lambda i: (0, 0)),
            pl.BlockSpec((1, 1), lambda i: (0, 0)),
            pl.BlockSpec((RB, 128), lambda i: (i, 0)),
            pl.BlockSpec((F, FC1), lambda i: (0, 0)),
            pl.BlockSpec((1, FC1), lambda i: (0, 0)),
            pl.BlockSpec((FC1, 1), lambda i: (0, 0)),
            pl.BlockSpec((1, 1), lambda i: (0, 0)),
        ],
        out_specs=pl.BlockSpec((G, 1), lambda i: (0, 0)),
        out_shape=jax.ShapeDtypeStruct((G, 1), jnp.float32),
        scratch_shapes=[pltpu.VMEM((128, F), jnp.float32),
                        pltpu.VMEM((128, F), jnp.float32)],
    )(out2, st, g, be, a, batch_b, Wf1, bf1, Wo, bo)


def kernel(x, edge_index, batch, W1, b1, g1, be1, a1, W2, b2, g2, be2, a2,
           Wf1, bf1, Wo, bo):
    f32 = jnp.float32
    x_p = jnp.pad(x, ((0, NP - N), (0, 0)))
    pad_e = jnp.full((EP - E,), DUMMY, jnp.int32)
    src = jnp.concatenate([edge_index[0], pad_e])
    dst_deg = jnp.concatenate([edge_index[1], pad_e])
    dst_agg = jnp.concatenate([edge_index[1], pad_e])
    src16 = src.reshape(16, NCH_AGG, CHA)
    dst16 = dst_agg.reshape(16, NCH_AGG, CHA)
    dst32 = dst_deg.reshape(32, NCH_DEG, CH)
    batch_p = jnp.pad(batch.astype(jnp.int32), (0, NP - N), constant_values=G)
    batch_b = jnp.broadcast_to(batch_p[:, None], (NP, 128))
    zeros_z = jnp.zeros((ZROWS_PT, HF), f32)
    zeros_d = jnp.zeros((ROWS_PT, 16), f32)
    e0rows = jnp.concatenate(
        [jnp.ones((CH, 1), f32), jnp.zeros((CH, 15), f32)], axis=1)

    b1r = b1.reshape(1, F)
    b2r = b2.reshape(1, F)
    g1r = g1.reshape(1, F)
    be1r = be1.reshape(1, F)
    g2r = g2.reshape(1, F)
    be2r = be2.reshape(1, F)
    a1r = a1.reshape(1, 1)
    a2r = a2.reshape(1, 1)
    bf1r = bf1.reshape(1, FC1)
    bor = bo.reshape(1, 1)

    d0, d1 = _sc_degree(dst32, e0rows, zeros_d)
    yl1, yr1 = _scaled_matmul(x_p, W1, d0, d1)
    zl1, zr1 = _sc_aggregate(yl1, yr1, src16, dst16, zeros_z)
    out1, st1 = _epilogue_stats(zl1, zr1, yl1, yr1, d0, d1, b1r)
    yl2, yr2 = _bn_matmul(out1, st1, g1r, be1r, a1r, W2, d0, d1)
    zl2, zr2 = _sc_aggregate(yl2, yr2, src16, dst16, zeros_z)
    out2, st2 = _epilogue_stats(zl2, zr2, yl2, yr2, d0, d1, b2r)
    return _final(out2, st2, g2r, be2r, a2r, batch_b, Wf1, bf1r, Wo, bor)
